# Initial kernel scaffold; baseline (speedup 1.0000x reference)
import functools

import jax
import jax.numpy as jnp
from jax import lax
from jax.experimental import pallas as pl
from jax.experimental.pallas import tpu as pltpu
from jax.experimental.pallas import tpu_sc as plsc

VOCAB = 1000000
DIM = 64
B_TOTAL = 16384 * 50
NC, NS = 2, 16
NW = NC * NS
PER_W = B_TOTAL // NW
SUB = 128
CHUNK = 512
K = CHUNK // SUB
NCHUNK = PER_W // CHUNK
NROW = PER_W // SUB

_mesh = plsc.VectorSubcoreMesh(core_axis_name="c", subcore_axis_name="s")


@functools.partial(
    pl.kernel,
    out_type=jax.ShapeDtypeStruct((B_TOTAL, DIM), jnp.float32),
    mesh=_mesh,
    scratch_types=[
        pltpu.VMEM((NROW, SUB), jnp.int32),
        pltpu.VMEM((CHUNK, DIM), jnp.float32),
        pltpu.SemaphoreType.DMA,
    ],
)
def _emb_kernel(weight_hbm, idx_hbm, out_hbm, idx_v, rows_v, sem):
    wid = lax.axis_index("s") * NC + lax.axis_index("c")
    base = wid * PER_W
    pltpu.sync_copy(idx_hbm.at[wid], idx_v)

    def chunk_body(c, carry):
        cps = []
        for j in range(K):
            cps.append(
                pltpu.async_copy(
                    weight_hbm.at[idx_v.at[c * K + j]],
                    rows_v.at[pl.ds(j * SUB, SUB)],
                    sem,
                )
            )
        for cp in cps:
            cp.wait()
        pltpu.sync_copy(rows_v, out_hbm.at[pl.ds(base + c * CHUNK, CHUNK)])
        return carry

    lax.fori_loop(0, NCHUNK, chunk_body, 0)


def kernel(x, weight):
    idx = x.reshape(NW, NROW, SUB).astype(jnp.int32)
    out = _emb_kernel(weight, idx)
    return out.reshape(x.shape[0], x.shape[1], DIM)


# speedup vs baseline: 1.8317x; 1.8317x over previous
import functools

import jax
import jax.numpy as jnp
from jax import lax
from jax.experimental import pallas as pl
from jax.experimental.pallas import tpu as pltpu
from jax.experimental.pallas import tpu_sc as plsc

VOCAB = 1000000
DIM = 64
B_TOTAL = 16384 * 50
NC, NS = 2, 16
NW = NC * NS
PER_W = B_TOTAL // NW
SUB = 128
CHUNK = 512
K = CHUNK // SUB
NCHUNK = PER_W // CHUNK
NROW = PER_W // SUB

_mesh = plsc.VectorSubcoreMesh(core_axis_name="c", subcore_axis_name="s")


@functools.partial(
    pl.kernel,
    out_type=jax.ShapeDtypeStruct((B_TOTAL, DIM), jnp.float32),
    mesh=_mesh,
    scratch_types=[
        pltpu.VMEM((NROW, SUB), jnp.int32),
        pltpu.VMEM((CHUNK, DIM), jnp.float32),
        pltpu.SemaphoreType.DMA,
    ],
    compiler_params=pltpu.CompilerParams(use_tc_tiling_on_sc=False),
)
def _emb_kernel(weight_hbm, idx_hbm, out_hbm, idx_v, rows_v, sem):
    wid = lax.axis_index("s") * NC + lax.axis_index("c")
    base = wid * PER_W
    pltpu.sync_copy(idx_hbm.at[wid], idx_v)

    def chunk_body(c, carry):
        cps = []
        for j in range(K):
            cps.append(
                pltpu.async_copy(
                    weight_hbm.at[idx_v.at[c * K + j]],
                    rows_v.at[pl.ds(j * SUB, SUB)],
                    sem,
                )
            )
        for cp in cps:
            cp.wait()
        pltpu.sync_copy(rows_v, out_hbm.at[pl.ds(base + c * CHUNK, CHUNK)])
        return carry

    lax.fori_loop(0, NCHUNK, chunk_body, 0)


def kernel(x, weight):
    idx = x.reshape(NW, NROW, SUB).astype(jnp.int32)
    out = _emb_kernel(weight, idx)
    return out.reshape(x.shape[0], x.shape[1], DIM)


# speedup vs baseline: 1.8639x; 1.0175x over previous
import functools

import jax
import jax.numpy as jnp
from jax import lax
from jax.experimental import pallas as pl
from jax.experimental.pallas import tpu as pltpu
from jax.experimental.pallas import tpu_sc as plsc

VOCAB = 1000000
DIM = 64
B_TOTAL = 16384 * 50
NC, NS = 2, 16
NW = NC * NS
PER_W = B_TOTAL // NW
SUB = 128
CHUNK = 512
K = CHUNK // SUB
NCHUNK = PER_W // CHUNK
NPAIR = NCHUNK // 2
NROW = PER_W // SUB

_mesh = plsc.VectorSubcoreMesh(core_axis_name="c", subcore_axis_name="s")


@functools.partial(
    pl.kernel,
    out_type=jax.ShapeDtypeStruct((B_TOTAL, DIM), jnp.float32),
    mesh=_mesh,
    scratch_types=[
        pltpu.VMEM((NROW, SUB), jnp.int32),
        pltpu.VMEM((CHUNK, DIM), jnp.float32),
        pltpu.VMEM((CHUNK, DIM), jnp.float32),
        pltpu.SemaphoreType.DMA,
        pltpu.SemaphoreType.DMA,
        pltpu.SemaphoreType.DMA,
        pltpu.SemaphoreType.DMA,
    ],
    compiler_params=pltpu.CompilerParams(use_tc_tiling_on_sc=False),
)
def _emb_kernel(weight_hbm, idx_hbm, out_hbm, idx_v, buf0, buf1,
                gsem0, gsem1, wsem0, wsem1):
    wid = lax.axis_index("s") * NC + lax.axis_index("c")
    base = wid * PER_W
    pltpu.sync_copy(idx_hbm.at[wid], idx_v)

    def fire(c, buf, gsem):
        for j in range(K):
            pltpu.async_copy(
                weight_hbm.at[idx_v.at[c * K + j]],
                buf.at[pl.ds(j * SUB, SUB)],
                gsem,
            )

    def drain(buf, gsem):
        pltpu.make_async_copy(weight_hbm.at[pl.ds(0, CHUNK)], buf, gsem).wait()

    def write(c, buf, wsem):
        return pltpu.async_copy(
            buf, out_hbm.at[pl.ds(base + c * CHUNK, CHUNK)], wsem)

    fire(0, buf0, gsem0)
    fire(1, buf1, gsem1)

    def pair_body(i, carry):
        c0 = 2 * i
        drain(buf0, gsem0)
        w0 = write(c0, buf0, wsem0)
        drain(buf1, gsem1)
        w1 = write(c0 + 1, buf1, wsem1)
        w0.wait()
        fire(c0 + 2, buf0, gsem0)
        w1.wait()
        fire(c0 + 3, buf1, gsem1)
        return carry

    lax.fori_loop(0, NPAIR - 1, pair_body, 0)

    drain(buf0, gsem0)
    w0 = write(NCHUNK - 2, buf0, wsem0)
    drain(buf1, gsem1)
    w1 = write(NCHUNK - 1, buf1, wsem1)
    w0.wait()
    w1.wait()


def kernel(x, weight):
    idx = x.reshape(NW, NROW, SUB).astype(jnp.int32)
    out = _emb_kernel(weight, idx)
    return out.reshape(x.shape[0], x.shape[1], DIM)
